# Initial kernel scaffold; baseline (speedup 1.0000x reference)
import jax
import jax.numpy as jnp
from jax.experimental import pallas as pl


def kernel(relative_bias_table, seq_len):
    raise NotImplementedError("write your pallas kernel here")


# speedup vs baseline: 1.5724x; 1.5724x over previous
import functools

import jax
import jax.numpy as jnp
import numpy as np
from jax import lax
from jax.experimental import pallas as pl
from jax.experimental.pallas import tpu as pltpu
from jax.experimental.pallas import tpu_sc as plsc

_MAX_REL = 7
_NUM_BUCKETS = 2 * _MAX_REL + 1
_NUM_HEADS = 16
_NUM_TILES = 32
_ROW_PAD = 80
_IDX_LEN = 3 * _ROW_PAD
_ZERO_ROW = _NUM_BUCKETS * _NUM_BUCKETS


def _host_indices() -> np.ndarray:
    idx = np.full((_NUM_TILES, _IDX_LEN), _ZERO_ROW, np.int32)
    for w in range(_NUM_TILES):
        rows = [0, 1, 2] if w == 0 else [2 * w + 1, 2 * w + 2]
        for li, orow in enumerate(rows):
            for c in range(65):
                if orow == 0 or c == 0:
                    continue
                i, j = orow - 1, c - 1
                dr = i // 8 - j // 8
                df = i % 8 - j % 8
                idx[w, li * _ROW_PAD + c] = (dr + _MAX_REL) * _NUM_BUCKETS + (
                    df + _MAX_REL)
    return idx


_IDX_HOST = _host_indices()


def _body(table_hbm, idx_hbm, out_hbm, table_v, idx_v, buf):
    wid = lax.axis_index("s") * 2 + lax.axis_index("c")
    pltpu.sync_copy(table_hbm, table_v)
    pltpu.sync_copy(idx_hbm.at[wid], idx_v)

    nchunks = jnp.where(wid == 0, 15, 10)

    def chunk(k, carry):
        li = k // 5
        c0 = (k % 5) * 16
        ivec = idx_v[pl.ds(k * 16, 16)]
        for h in range(_NUM_HEADS):
            g = plsc.load_gather(table_v, [ivec, jnp.full((16,), h, jnp.int32)])
            e = jnp.exp(jnp.abs(g) * -2.0)
            val = jnp.sign(g) * ((2.0 - 2.0 * e) / (1.0 + e))
            buf[h, li, pl.ds(c0, 16)] = val
        return carry

    lax.fori_loop(0, nchunks, chunk, 0)

    @pl.when(wid == 0)
    def _():
        pltpu.sync_copy(buf, out_hbm.at[:, pl.ds(0, 3), :])

    @pl.when(wid != 0)
    def _():
        pltpu.sync_copy(buf.at[:, :2, :],
                        out_hbm.at[:, pl.ds(2 * wid + 1, 2), :])


@jax.jit
def _run(table2d):
    mesh = plsc.VectorSubcoreMesh(core_axis_name="c", subcore_axis_name="s")
    out = pl.kernel(
        _body,
        out_type=jax.ShapeDtypeStruct((_NUM_HEADS, 65, _ROW_PAD), jnp.float32),
        mesh=mesh,
        compiler_params=pltpu.CompilerParams(use_tc_tiling_on_sc=False,
                                             needs_layout_passes=False),
        scratch_types=[
            pltpu.VMEM((_ZERO_ROW + 1, _NUM_HEADS), jnp.float32),
            pltpu.VMEM((_IDX_LEN,), jnp.int32),
            pltpu.VMEM((_NUM_HEADS, 3, _ROW_PAD), jnp.float32),
        ],
    )(table2d, jnp.asarray(_IDX_HOST))
    return out[:, :, :65].reshape(1, _NUM_HEADS, 65, 65)


def kernel(relative_bias_table, seq_len):
    del seq_len
    table2d = jnp.concatenate(
        [relative_bias_table.reshape(_ZERO_ROW, _NUM_HEADS),
         jnp.zeros((1, _NUM_HEADS), jnp.float32)], axis=0)
    return _run(table2d)
